# Initial kernel scaffold; baseline (speedup 1.0000x reference)
import jax
import jax.numpy as jnp
import numpy as np
from jax.experimental import pallas as pl
from jax.experimental.pallas import tpu as pltpu

T = 2048
HID = 768
H, D = 12, 64
FW = 64
A_SAL, B_SAL = 1.0, 0.25
ALPHA = 0.1
TAU = max(FW / 4.0, 1.0)
KK = 48
G_GLOB, T_TELE = 4, 2
EX = G_GLOB + T_TELE + 2
SCALE = 1.0 / np.sqrt(D)
BQ = 256
BAND = 384
BIG = 1e30

_TELE = np.round(np.linspace(0.0, T - 1.0, T_TELE + 2)[1:-1]).astype(np.int32)


def _qkv_kernel(x_ref, wq_ref, bq_ref, wk_ref, bk_ref, wv_ref, bv_ref,
                q_ref, k_ref, v_ref):
    x = x_ref[...]
    q_ref[...] = jnp.dot(x, wq_ref[...], preferred_element_type=jnp.float32) + bq_ref[...]
    k_ref[...] = jnp.dot(x, wk_ref[...], preferred_element_type=jnp.float32) + bk_ref[...]
    v_ref[...] = jnp.dot(x, wv_ref[...], preferred_element_type=jnp.float32) + bv_ref[...]


def _route_kernel(k_ref, g_ref):
    k = k_ref[...]
    kprev = jnp.concatenate([k[:1], k[:-1]], axis=0)
    dk = k - kprev
    jrow = jax.lax.broadcasted_iota(jnp.int32, (HID, 128), 0)
    hcol = jax.lax.broadcasted_iota(jnp.int32, (HID, 128), 1)
    sel = (jrow // D == hcol).astype(jnp.float32)
    kn = jnp.sqrt(jnp.dot(k * k, sel, preferred_element_type=jnp.float32))
    dn = jnp.sqrt(jnp.dot(dk * dk, sel, preferred_element_type=jnp.float32))
    sal = A_SAL * kn + B_SAL * dn
    salb = sal.reshape(G_GLOB, T // G_GLOB, 128)
    mx = jnp.max(salb, axis=1, keepdims=True)
    ii = jax.lax.broadcasted_iota(jnp.int32, salb.shape, 1)
    idx = jnp.min(jnp.where(salb == mx, ii, T), axis=1)
    goff = jax.lax.broadcasted_iota(jnp.int32, (G_GLOB, 128), 0) * (T // G_GLOB)
    g_ref[...] = jnp.concatenate(
        [idx + goff, jnp.zeros((8 - G_GLOB, 128), jnp.int32)], axis=0)


def _attn_kernel(eidx_ref, q_ref, k_ref, v_ref, ef_ref, o_ref, ke_scr, ve_scr):
    h = pl.program_id(0)
    qb = pl.program_id(1)
    t0 = qb * BQ

    ke_scr[...] = jnp.zeros((128, D), jnp.float32)
    ve_scr[...] = jnp.zeros((128, D), jnp.float32)
    for j in range(EX):
        e = eidx_ref[h * EX + j]
        ke_scr[pl.ds(j, 1), :] = k_ref[pl.ds(e, 1), :]
        ve_scr[pl.ds(j, 1), :] = v_ref[pl.ds(e, 1), :]

    q = q_ref[...]
    s0 = jnp.clip(t0 - 64, 0, T - BAND)
    kb = k_ref[pl.ds(s0, BAND), :]
    vb = v_ref[pl.ds(s0, BAND), :]

    sw = jax.lax.dot_general(q, kb, (((1,), (1,)), ((), ())),
                             preferred_element_type=jnp.float32) * SCALE
    rows = t0 + jax.lax.broadcasted_iota(jnp.int32, (BQ, BAND), 0)
    cols = s0 + jax.lax.broadcasted_iota(jnp.int32, (BQ, BAND), 1)
    starts = jnp.clip(rows - FW // 2, 0, T - FW)
    valid = (cols >= starts) & (cols < starts + FW)
    dist = jnp.abs(cols - rows).astype(jnp.float32)
    sb = sw - (ALPHA / TAU) * dist

    work = jnp.where(valid, sb, BIG)
    for _ in range(FW - KK):
        m = jnp.min(work, axis=1, keepdims=True)
        work = jnp.where(work == m, BIG, work)
    kept = work < (BIG * 0.5)
    swin = jnp.where(kept, sb, -BIG)

    se = jax.lax.dot_general(q, ke_scr[...], (((1,), (1,)), ((), ())),
                             preferred_element_type=jnp.float32) * SCALE
    ef = ef_ref[0]
    trow = (rows[:, :128]).astype(jnp.float32)
    lane = jax.lax.broadcasted_iota(jnp.int32, (BQ, 128), 1)
    se = jnp.where(lane < EX, se - (ALPHA / TAU) * jnp.abs(ef - trow), -BIG)

    mrow = jnp.maximum(jnp.max(swin, axis=1, keepdims=True),
                       jnp.max(se, axis=1, keepdims=True))
    pw = jnp.where(kept, jnp.exp(swin - mrow), 0.0)
    pe = jnp.where(lane < EX, jnp.exp(se - mrow), 0.0)
    denom = (jnp.sum(pw, axis=1, keepdims=True)
             + jnp.sum(pe, axis=1, keepdims=True))
    acc = (jnp.dot(pw, vb, preferred_element_type=jnp.float32)
           + jnp.dot(pe, ve_scr[...], preferred_element_type=jnp.float32))
    o_ref[...] = acc / denom


@jax.jit
def _run(x, Wq, bq, Wk, bk, Wv, bv):
    x2 = x.reshape(T, HID)

    q2, k2, v2 = pl.pallas_call(
        _qkv_kernel,
        grid=(T // BQ,),
        in_specs=[
            pl.BlockSpec((BQ, HID), lambda i: (i, 0)),
            pl.BlockSpec((HID, HID), lambda i: (0, 0)),
            pl.BlockSpec((1, HID), lambda i: (0, 0)),
            pl.BlockSpec((HID, HID), lambda i: (0, 0)),
            pl.BlockSpec((1, HID), lambda i: (0, 0)),
            pl.BlockSpec((HID, HID), lambda i: (0, 0)),
            pl.BlockSpec((1, HID), lambda i: (0, 0)),
        ],
        out_specs=[
            pl.BlockSpec((BQ, HID), lambda i: (i, 0)),
            pl.BlockSpec((BQ, HID), lambda i: (i, 0)),
            pl.BlockSpec((BQ, HID), lambda i: (i, 0)),
        ],
        out_shape=[jax.ShapeDtypeStruct((T, HID), jnp.float32)] * 3,
    )(x2, Wq, bq.reshape(1, HID), Wk, bk.reshape(1, HID), Wv, bv.reshape(1, HID))

    g = pl.pallas_call(
        _route_kernel,
        out_shape=jax.ShapeDtypeStruct((8, 128), jnp.int32),
    )(k2)

    g_idx = g[:G_GLOB, :H].T
    extras = jnp.concatenate([
        g_idx,
        jnp.broadcast_to(jnp.asarray(_TELE)[None, :], (H, T_TELE)),
        jnp.zeros((H, 1), jnp.int32),
        jnp.full((H, 1), T - 1, jnp.int32),
    ], axis=1)
    eflat = extras.reshape(H * EX)
    ef32 = jnp.zeros((H, 1, 128), jnp.float32).at[:, 0, :EX].set(
        extras.astype(jnp.float32))

    out = pl.pallas_call(
        _attn_kernel,
        grid_spec=pltpu.PrefetchScalarGridSpec(
            num_scalar_prefetch=1,
            grid=(H, T // BQ),
            in_specs=[
                pl.BlockSpec((BQ, D), lambda h, qb, e: (qb, h)),
                pl.BlockSpec((T, D), lambda h, qb, e: (0, h)),
                pl.BlockSpec((T, D), lambda h, qb, e: (0, h)),
                pl.BlockSpec((1, 1, 128), lambda h, qb, e: (h, 0, 0)),
            ],
            out_specs=pl.BlockSpec((BQ, D), lambda h, qb, e: (qb, h)),
            scratch_shapes=[
                pltpu.VMEM((128, D), jnp.float32),
                pltpu.VMEM((128, D), jnp.float32),
            ],
        ),
        out_shape=jax.ShapeDtypeStruct((T, HID), jnp.float32),
    )(eflat, q2, k2, v2, ef32)

    return out.reshape(1, T, HID)


def kernel(hidden_states, Wq, bq, Wk, bk, Wv, bv):
    return _run(hidden_states, Wq, bq, Wk, bk, Wv, bv)


# speedup vs baseline: 103.7674x; 103.7674x over previous
import jax
import jax.numpy as jnp
import numpy as np
from jax.experimental import pallas as pl
from jax.experimental.pallas import tpu as pltpu

T = 2048
HID = 768
H, D = 12, 64
FW = 64
A_SAL, B_SAL = 1.0, 0.25
ALPHA = 0.1
TAU = max(FW / 4.0, 1.0)
KK = 48
G_GLOB, T_TELE = 4, 2
EX = G_GLOB + T_TELE + 2
SCALE = 1.0 / np.sqrt(D)
BQ = 256
BAND = 384
BIG = 1e30

_TELE = np.round(np.linspace(0.0, T - 1.0, T_TELE + 2)[1:-1]).astype(np.int32)


def _qkv_kernel(x_ref, wq_ref, bq_ref, wk_ref, bk_ref, wv_ref, bv_ref,
                q_ref, k_ref, v_ref):
    x = x_ref[...]
    q_ref[0] = jnp.dot(x, wq_ref[0], preferred_element_type=jnp.float32, precision=jax.lax.Precision.DEFAULT) + bq_ref[0]
    k_ref[0] = jnp.dot(x, wk_ref[0], preferred_element_type=jnp.float32, precision=jax.lax.Precision.DEFAULT) + bk_ref[0]
    v_ref[0] = jnp.dot(x, wv_ref[0], preferred_element_type=jnp.float32, precision=jax.lax.Precision.DEFAULT) + bv_ref[0]


def _route_kernel(k_ref, g_ref):
    k = k_ref[0]
    kprev = jnp.concatenate([k[:1], k[:-1]], axis=0)
    dk = k - kprev
    kn = jnp.sqrt(jnp.sum(k * k, axis=1, keepdims=True))
    dn = jnp.sqrt(jnp.sum(dk * dk, axis=1, keepdims=True))
    sal = A_SAL * kn + B_SAL * dn
    salb = sal.reshape(G_GLOB, T // G_GLOB, 1)
    mx = jnp.max(salb, axis=1, keepdims=True)
    ii = jax.lax.broadcasted_iota(jnp.int32, salb.shape, 1)
    idx = jnp.min(jnp.where(salb == mx, ii, T), axis=1)
    goff = jax.lax.broadcasted_iota(jnp.int32, (G_GLOB, 1), 0) * (T // G_GLOB)
    gfull = jnp.broadcast_to(idx + goff, (G_GLOB, 128))
    g_ref[0] = jnp.concatenate(
        [gfull, jnp.zeros((8 - G_GLOB, 128), jnp.int32)], axis=0)


def _attn_kernel(eidx_ref, q_ref, k_ref, v_ref, ef_ref, o_ref, ke_scr, ve_scr):
    h = pl.program_id(0)
    qb = pl.program_id(1)
    t0 = qb * BQ

    ke_scr[...] = jnp.zeros((128, D), jnp.float32)
    ve_scr[...] = jnp.zeros((128, D), jnp.float32)
    for j in range(EX):
        e = eidx_ref[h * EX + j]
        ke_scr[pl.ds(j, 1), :] = k_ref[0, pl.ds(e, 1), :]
        ve_scr[pl.ds(j, 1), :] = v_ref[0, pl.ds(e, 1), :]

    q = q_ref[0]
    s0 = jnp.clip(t0 - 64, 0, T - BAND)
    kb = k_ref[0, pl.ds(s0, BAND), :]
    vb = v_ref[0, pl.ds(s0, BAND), :]

    sw = jax.lax.dot_general(q, kb, (((1,), (1,)), ((), ())),
                             preferred_element_type=jnp.float32, precision=jax.lax.Precision.HIGHEST) * SCALE
    rows = t0 + jax.lax.broadcasted_iota(jnp.int32, (BQ, BAND), 0)
    cols = s0 + jax.lax.broadcasted_iota(jnp.int32, (BQ, BAND), 1)
    starts = jnp.clip(rows - FW // 2, 0, T - FW)
    valid = (cols >= starts) & (cols < starts + FW)
    dist = jnp.abs(cols - rows).astype(jnp.float32)
    sb = sw - (ALPHA / TAU) * dist

    work = jnp.where(valid, sb, BIG)
    for _ in range(FW - KK):
        m = jnp.min(work, axis=1, keepdims=True)
        work = jnp.where(work == m, BIG, work)
    kept = work < (BIG * 0.5)
    swin = jnp.where(kept, sb, -BIG)

    se = jax.lax.dot_general(q, ke_scr[...], (((1,), (1,)), ((), ())),
                             preferred_element_type=jnp.float32, precision=jax.lax.Precision.HIGHEST) * SCALE
    ef = ef_ref[0]
    trow = (rows[:, :128]).astype(jnp.float32)
    lane = jax.lax.broadcasted_iota(jnp.int32, (BQ, 128), 1)
    se = jnp.where(lane < EX, se - (ALPHA / TAU) * jnp.abs(ef - trow), -BIG)

    mrow = jnp.maximum(jnp.max(swin, axis=1, keepdims=True),
                       jnp.max(se, axis=1, keepdims=True))
    pw = jnp.where(kept, jnp.exp(swin - mrow), 0.0)
    pe = jnp.where(lane < EX, jnp.exp(se - mrow), 0.0)
    denom = (jnp.sum(pw, axis=1, keepdims=True)
             + jnp.sum(pe, axis=1, keepdims=True))
    acc = (jnp.dot(pw, vb, preferred_element_type=jnp.float32, precision=jax.lax.Precision.HIGHEST)
           + jnp.dot(pe, ve_scr[...], preferred_element_type=jnp.float32, precision=jax.lax.Precision.HIGHEST))
    o_ref[0] = acc / denom


@jax.jit
def _run(x, Wq, bq, Wk, bk, Wv, bv):
    x2 = x.reshape(T, HID)
    Wq3 = Wq.reshape(HID, H, D).transpose(1, 0, 2)
    Wk3 = Wk.reshape(HID, H, D).transpose(1, 0, 2)
    Wv3 = Wv.reshape(HID, H, D).transpose(1, 0, 2)
    bq3 = bq.reshape(H, 1, D)
    bk3 = bk.reshape(H, 1, D)
    bv3 = bv.reshape(H, 1, D)

    wspec = pl.BlockSpec((1, HID, D), lambda i, h: (h, 0, 0))
    bspec = pl.BlockSpec((1, 1, D), lambda i, h: (h, 0, 0))
    ospec = pl.BlockSpec((1, BQ, D), lambda i, h: (h, i, 0))
    q3, k3, v3 = pl.pallas_call(
        _qkv_kernel,
        grid=(T // BQ, H),
        in_specs=[
            pl.BlockSpec((BQ, HID), lambda i, h: (i, 0)),
            wspec, bspec, wspec, bspec, wspec, bspec,
        ],
        out_specs=[ospec, ospec, ospec],
        out_shape=[jax.ShapeDtypeStruct((H, T, D), jnp.float32)] * 3,
    )(x2, Wq3, bq3, Wk3, bk3, Wv3, bv3)

    g3 = pl.pallas_call(
        _route_kernel,
        grid=(H,),
        in_specs=[pl.BlockSpec((1, T, D), lambda h: (h, 0, 0))],
        out_specs=pl.BlockSpec((1, 8, 128), lambda h: (h, 0, 0)),
        out_shape=jax.ShapeDtypeStruct((H, 8, 128), jnp.int32),
    )(k3)

    g_idx = g3[:, :G_GLOB, 0]
    extras = jnp.concatenate([
        g_idx,
        jnp.broadcast_to(jnp.asarray(_TELE)[None, :], (H, T_TELE)),
        jnp.zeros((H, 1), jnp.int32),
        jnp.full((H, 1), T - 1, jnp.int32),
    ], axis=1)
    eflat = extras.reshape(H * EX)
    ef32 = jnp.zeros((H, 1, 128), jnp.float32).at[:, 0, :EX].set(
        extras.astype(jnp.float32))

    out3 = pl.pallas_call(
        _attn_kernel,
        grid_spec=pltpu.PrefetchScalarGridSpec(
            num_scalar_prefetch=1,
            grid=(H, T // BQ),
            in_specs=[
                pl.BlockSpec((1, BQ, D), lambda h, qb, e: (h, qb, 0)),
                pl.BlockSpec((1, T, D), lambda h, qb, e: (h, 0, 0)),
                pl.BlockSpec((1, T, D), lambda h, qb, e: (h, 0, 0)),
                pl.BlockSpec((1, 1, 128), lambda h, qb, e: (h, 0, 0)),
            ],
            out_specs=pl.BlockSpec((1, BQ, D), lambda h, qb, e: (h, qb, 0)),
            scratch_shapes=[
                pltpu.VMEM((128, D), jnp.float32),
                pltpu.VMEM((128, D), jnp.float32),
            ],
        ),
        out_shape=jax.ShapeDtypeStruct((H, T, D), jnp.float32),
    )(eflat, q3, k3, v3, ef32)

    return out3.transpose(1, 0, 2).reshape(1, T, HID)


def kernel(hidden_states, Wq, bq, Wk, bk, Wv, bv):
    return _run(hidden_states, Wq, bq, Wk, bk, Wv, bv)
